# baseline (speedup 1.0000x reference)
import functools

import jax
import jax.numpy as jnp
from jax import lax
from jax.experimental import pallas as pl
from jax.experimental.pallas import tpu as pltpu
from jax.experimental.pallas import tpu_sc as plsc

S = 512
BATCH = 4
SS = S * S
OFF = S + 1
NKEY = BATCH * SS
DUMMY = NKEY + 2 * S + 2
TBL = 16 * 36 * 2048
TAPS = tuple((dx, dy) for dx in (-1, 0, 1) for dy in (-1, 0, 1))

C2 = 32
R2 = C2 * 9
RP = 384


def _gather_taps(npad, nin, n):
  info = plsc.get_sparse_core_info()
  nc, ns = info.num_cores, info.num_subcores
  nw = nc * ns
  per_w = npad // nw
  chunks = per_w // C2
  per_tile = TBL // ns
  schunks = npad // ns // C2
  zrow = n

  mesh = plsc.VectorSubcoreMesh(core_axis_name="c", subcore_axis_name="s")

  @functools.partial(
      pl.kernel, mesh=mesh,
      out_type=jax.ShapeDtypeStruct((9, npad, nin), jnp.float32),
      scratch_types=[
          pltpu.VMEM((1024,), jnp.int32),
          pltpu.VMEM((C2,), jnp.int32),
          pltpu.VMEM((C2,), jnp.int32),
          pltpu.VMEM((C2,), jnp.int32),
          pltpu.VMEM((C2,), jnp.int32),
          pltpu.VMEM((C2,), jnp.int32),
          pltpu.VMEM((RP,), jnp.int32),
          pltpu.VMEM((RP,), jnp.int32),
          pltpu.VMEM((RP, nin), jnp.float32),
          pltpu.VMEM_SHARED((TBL,), jnp.int32),
          pltpu.SemaphoreType.DMA,
          pltpu.SemaphoreType.DMA,
      ],
  )
  def gather(b_hbm, x_hbm, y_hbm, f_hbm, g_hbm,
             cbuf, bb, xb, yb, si, sv, tix, fix, gbuf, tshared, sem1, sem2):
    cid = lax.axis_index("c")
    sid = lax.axis_index("s")
    wid = sid * nc + cid
    iota = lax.iota(jnp.int32, 16)

    zr16 = jnp.full((16,), zrow, jnp.int32)
    for j in range(1024 // 16):
      cbuf[pl.ds(j * 16, 16)] = zr16

    def init_body(r, carry):
      pltpu.sync_copy(cbuf, tshared.at[pl.ds(sid * per_tile + r * 1024, 1024)])
      return carry

    lax.fori_loop(0, per_tile // 1024, init_body, 0)
    plsc.subcore_barrier()

    def scat_body(ch, carry):
      base = sid * (schunks * C2) + ch * C2
      pltpu.sync_copy(b_hbm.at[pl.ds(base, C2)], bb)
      pltpu.sync_copy(x_hbm.at[pl.ds(base, C2)], xb)
      pltpu.sync_copy(y_hbm.at[pl.ds(base, C2)], yb)
      for j in range(C2 // 16):
        bv = bb[pl.ds(j * 16, 16)]
        xv = xb[pl.ds(j * 16, 16)]
        yv = yb[pl.ds(j * 16, 16)]
        key = bv * SS + xv * S + yv
        rowid = base + j * 16 + iota
        live = rowid < n
        si[pl.ds(j * 16, 16)] = jnp.where(live, key + OFF, DUMMY)
        sv[pl.ds(j * 16, 16)] = jnp.where(live, rowid, zrow)
      pltpu.sync_copy(sv, tshared.at[si])
      return carry

    lax.fori_loop(0, schunks, scat_body, 0)
    plsc.subcore_barrier()

    dummy16 = jnp.full((16,), DUMMY, jnp.int32)
    for t in range((RP - R2) // 16):
      tix[pl.ds(R2 + t * 16, 16)] = dummy16

    def chunk_body(ch, carry):
      base = wid * per_w + ch * C2
      pltpu.sync_copy(b_hbm.at[pl.ds(base, C2)], bb)
      pltpu.sync_copy(x_hbm.at[pl.ds(base, C2)], xb)
      pltpu.sync_copy(y_hbm.at[pl.ds(base, C2)], yb)
      for j in range(C2 // 16):
        bv = bb[pl.ds(j * 16, 16)]
        xv = xb[pl.ds(j * 16, 16)]
        yv = yb[pl.ds(j * 16, 16)]
        key = bv * SS + xv * S + yv + OFF
        for k, (dx, dy) in enumerate(TAPS):
          nk = key + (dx * S + dy)
          conds = []
          if dx < 0:
            conds.append(xv > 0)
          if dx > 0:
            conds.append(xv < S - 1)
          if dy < 0:
            conds.append(yv > 0)
          if dy > 0:
            conds.append(yv < S - 1)
          if conds:
            ok = conds[0]
            for c in conds[1:]:
              ok = ok & c
            nk = jnp.where(ok, nk, DUMMY)
          tix[pl.ds(k * C2 + j * 16, 16)] = nk
      hs = [pltpu.async_copy(tshared.at[tix.at[pl.ds(q * 128, 128)]],
                             fix.at[pl.ds(q * 128, 128)], sem1)
            for q in range(RP // 128)]
      for h in hs:
        h.wait()
      hs = [pltpu.async_copy(f_hbm.at[fix.at[pl.ds(q * 128, 128)]],
                             gbuf.at[pl.ds(q * 128, 128)], sem2)
            for q in range(RP // 128)]
      for h in hs:
        h.wait()
      for k in range(9):
        pltpu.sync_copy(gbuf.at[pl.ds(k * C2, C2)],
                        g_hbm.at[k, pl.ds(base, C2)])
      return carry

    lax.fori_loop(0, chunks, chunk_body, 0)

  return gather


def _tap_matmul(nt, bn, nin, nout):
  def body(g_ref, w_ref, b_ref, o_ref):
    acc = b_ref[...].astype(jnp.float32)
    for k in range(9):
      acc = acc + jnp.dot(g_ref[k], w_ref[k],
                          preferred_element_type=jnp.float32)
    o_ref[...] = jnp.maximum(acc, 0.0)

  return pl.pallas_call(
      body,
      grid=(nt // bn,),
      in_specs=[
          pl.BlockSpec((9, bn, nin), lambda i: (0, i, 0)),
          pl.BlockSpec((9, nin, nout), lambda i: (0, 0, 0)),
          pl.BlockSpec((1, nout), lambda i: (0, 0)),
      ],
      out_specs=pl.BlockSpec((bn, nout), lambda i: (i, 0)),
      out_shape=jax.ShapeDtypeStruct((nt, nout), jnp.float32),
  )


def kernel(features, coordinates, W, bias):
  n, nin = features.shape
  nout = W.shape[2]
  npad = -(-n // 2048) * 2048
  pad = npad - n

  coords = coordinates.astype(jnp.int32)
  bcol = jnp.pad(coords[:, 0], (0, pad))
  xcol = jnp.pad(coords[:, 1], (0, pad))
  ycol = jnp.pad(coords[:, 2], (0, pad))
  fext = jnp.concatenate(
      [features, jnp.zeros((8, nin), features.dtype)], axis=0)

  g = _gather_taps(npad, nin, n)(bcol, xcol, ycol, fext)

  bn = 512
  nt = -(-n // bn) * bn
  out = _tap_matmul(nt, bn, nin, nout)(g, W, bias.reshape(1, nout))
  return out[:n]


# speedup vs baseline: 1.1176x; 1.1176x over previous
import functools

import jax
import jax.numpy as jnp
from jax import lax
from jax.experimental import pallas as pl
from jax.experimental.pallas import tpu as pltpu
from jax.experimental.pallas import tpu_sc as plsc

S = 512
BATCH = 4
SS = S * S
OFF = S + 1
NKEY = BATCH * SS
DUMMY = NKEY + 2 * S + 16
FTBL = 16 * 66 * 1024
TSLOT = FTBL // 128
TAPS = tuple((dx, dy) for dx in (-1, 0, 1) for dy in (-1, 0, 1))

C1 = 128
C2 = 64
R2 = C2 * 9
RP = 640


def _build_table(npad, n, zrow):
  ns = 16
  chunks = npad // ns // C1
  per_w = FTBL // ns

  mesh = plsc.VectorSubcoreMesh(core_axis_name="c", subcore_axis_name="s",
                                num_cores=1)

  @functools.partial(
      pl.kernel, mesh=mesh,
      out_type=jax.ShapeDtypeStruct((FTBL,), jnp.int32),
      scratch_types=[
          pltpu.VMEM((1024,), jnp.int32),
          pltpu.VMEM((C1,), jnp.int32),
          pltpu.VMEM((C1,), jnp.int32),
          pltpu.VMEM((C1,), jnp.int32),
          pltpu.VMEM((C1,), jnp.int32),
          pltpu.VMEM((C1,), jnp.int32),
          pltpu.SemaphoreType.DMA,
      ],
  )
  def build(b_hbm, x_hbm, y_hbm, table_hbm, cbuf, bb, xb, yb, si, sv, sem):
    wid = lax.axis_index("s")
    zr16 = jnp.full((16,), zrow, jnp.int32)
    for j in range(1024 // 16):
      cbuf[pl.ds(j * 16, 16)] = zr16

    def init_body(r, carry):
      pltpu.sync_copy(cbuf, table_hbm.at[pl.ds(wid * per_w + r * 1024, 1024)])
      return carry

    lax.fori_loop(0, per_w // 1024, init_body, 0)
    plsc.subcore_barrier()

    iota = lax.iota(jnp.int32, 16)

    def chunk_body(ch, carry):
      base = wid * (chunks * C1) + ch * C1
      pltpu.sync_copy(b_hbm.at[pl.ds(base, C1)], bb)
      pltpu.sync_copy(x_hbm.at[pl.ds(base, C1)], xb)
      pltpu.sync_copy(y_hbm.at[pl.ds(base, C1)], yb)
      for j in range(C1 // 16):
        bv = bb[pl.ds(j * 16, 16)]
        xv = xb[pl.ds(j * 16, 16)]
        yv = yb[pl.ds(j * 16, 16)]
        key = bv * SS + xv * S + yv
        rowid = base + j * 16 + iota
        live = rowid < n
        si[pl.ds(j * 16, 16)] = jnp.where(live, key + OFF, DUMMY)
        sv[pl.ds(j * 16, 16)] = jnp.where(live, rowid, zrow)
      pltpu.async_copy(sv, table_hbm.at[si], sem).wait()
      return carry

    lax.fori_loop(0, chunks, chunk_body, 0)

  return build


def _tap_lines(npad):
  info = plsc.get_sparse_core_info()
  nc, ns = info.num_cores, info.num_subcores
  nw = nc * ns
  per_w = npad // nw
  chunks = per_w // C2

  mesh = plsc.VectorSubcoreMesh(core_axis_name="c", subcore_axis_name="s")

  @functools.partial(
      pl.kernel, mesh=mesh,
      out_type=(jax.ShapeDtypeStruct((npad * 9, 128), jnp.int32),
                jax.ShapeDtypeStruct((npad * 9,), jnp.int32)),
      scratch_types=[
          pltpu.VMEM((C2,), jnp.int32),
          pltpu.VMEM((C2,), jnp.int32),
          pltpu.VMEM((C2,), jnp.int32),
          pltpu.VMEM((RP,), jnp.int32),
          pltpu.VMEM((RP,), jnp.int32),
          pltpu.VMEM((RP, 128), jnp.int32),
          pltpu.SemaphoreType.DMA,
      ],
  )
  def lines(table_hbm, b_hbm, x_hbm, y_hbm, lines_hbm, pw_hbm,
            bb, xb, yb, trow, pwb, frows, sem1):
    wid = lax.axis_index("s") * nc + lax.axis_index("c")
    drow16 = jnp.full((16,), DUMMY >> 7, jnp.int32)
    zero16 = jnp.zeros((16,), jnp.int32)
    for t in range((RP - R2) // 16):
      trow[pl.ds(R2 + t * 16, 16)] = drow16
      pwb[pl.ds(R2 + t * 16, 16)] = zero16

    def chunk_body(ch, carry):
      base = wid * per_w + ch * C2
      pltpu.sync_copy(b_hbm.at[pl.ds(base, C2)], bb)
      pltpu.sync_copy(x_hbm.at[pl.ds(base, C2)], xb)
      pltpu.sync_copy(y_hbm.at[pl.ds(base, C2)], yb)
      for j in range(C2 // 16):
        bv = bb[pl.ds(j * 16, 16)]
        xv = xb[pl.ds(j * 16, 16)]
        yv = yb[pl.ds(j * 16, 16)]
        key = bv * SS + xv * S + yv + OFF
        for k, (dx, dy) in enumerate(TAPS):
          nk = key + (dx * S + dy)
          conds = []
          if dx < 0:
            conds.append(xv > 0)
          if dx > 0:
            conds.append(xv < S - 1)
          if dy < 0:
            conds.append(yv > 0)
          if dy > 0:
            conds.append(yv < S - 1)
          if conds:
            ok = conds[0]
            for c in conds[1:]:
              ok = ok & c
            nk = jnp.where(ok, nk, DUMMY)
          trow[pl.ds(k * C2 + j * 16, 16)] = nk >> 7
          pwb[pl.ds(k * C2 + j * 16, 16)] = nk & 127
      hs = [pltpu.async_copy(table_hbm.at[trow.at[pl.ds(q * 128, 128)]],
                             frows.at[pl.ds(q * 128, 128)], sem1)
            for q in range(RP // 128)]
      for h in hs:
        h.wait()
      pltpu.sync_copy(frows.at[pl.ds(0, R2)],
                      lines_hbm.at[pl.ds(base * 9, R2)])
      pltpu.sync_copy(pwb.at[pl.ds(0, R2)], pw_hbm.at[pl.ds(base * 9, R2)])
      return carry

    lax.fori_loop(0, chunks, chunk_body, 0)

  return lines


def _extract(nrows, rb):
  def body(l_ref, p_ref, o_ref):
    sel = lax.broadcasted_iota(jnp.int32, (rb, 128), 1) == p_ref[...]
    o_ref[...] = jnp.sum(jnp.where(sel, l_ref[...], 0), axis=1, keepdims=True)

  return pl.pallas_call(
      body,
      grid=(nrows // rb,),
      in_specs=[
          pl.BlockSpec((rb, 128), lambda i: (i, 0)),
          pl.BlockSpec((rb, 1), lambda i: (i, 0)),
      ],
      out_specs=pl.BlockSpec((rb, 1), lambda i: (i, 0)),
      out_shape=jax.ShapeDtypeStruct((nrows, 1), jnp.int32),
  )


def _gather_rows(npad, nin):
  info = plsc.get_sparse_core_info()
  nc, ns = info.num_cores, info.num_subcores
  nw = nc * ns
  per_w = npad // nw
  chunks = per_w // C2

  mesh = plsc.VectorSubcoreMesh(core_axis_name="c", subcore_axis_name="s")

  @functools.partial(
      pl.kernel, mesh=mesh,
      out_type=jax.ShapeDtypeStruct((9, npad, nin), jnp.float32),
      scratch_types=[
          pltpu.VMEM((RP,), jnp.int32),
          pltpu.VMEM((RP, nin), jnp.float32),
          pltpu.SemaphoreType.DMA,
      ],
  )
  def gather(fix_hbm, f_hbm, g_hbm, fixb, gbuf, sem2):
    wid = lax.axis_index("s") * nc + lax.axis_index("c")
    zr16 = jnp.zeros((16,), jnp.int32)
    for t in range((RP - R2) // 16):
      fixb[pl.ds(R2 + t * 16, 16)] = zr16

    def chunk_body(ch, carry):
      base = wid * per_w + ch * C2
      pltpu.sync_copy(fix_hbm.at[pl.ds(base * 9, R2)], fixb.at[pl.ds(0, R2)])
      hs = [pltpu.async_copy(f_hbm.at[fixb.at[pl.ds(q * 128, 128)]],
                             gbuf.at[pl.ds(q * 128, 128)], sem2)
            for q in range(RP // 128)]
      for h in hs:
        h.wait()
      for k in range(9):
        pltpu.sync_copy(gbuf.at[pl.ds(k * C2, C2)],
                        g_hbm.at[k, pl.ds(base, C2)])
      return carry

    lax.fori_loop(0, chunks, chunk_body, 0)

  return gather


def _tap_matmul(nt, bn, nin, nout):
  def body(g_ref, w_ref, b_ref, o_ref):
    acc = b_ref[...].astype(jnp.float32)
    for k in range(9):
      acc = acc + jnp.dot(g_ref[k], w_ref[k],
                          preferred_element_type=jnp.float32)
    o_ref[...] = jnp.maximum(acc, 0.0)

  return pl.pallas_call(
      body,
      grid=(nt // bn,),
      in_specs=[
          pl.BlockSpec((9, bn, nin), lambda i: (0, i, 0)),
          pl.BlockSpec((9, nin, nout), lambda i: (0, 0, 0)),
          pl.BlockSpec((1, nout), lambda i: (0, 0)),
      ],
      out_specs=pl.BlockSpec((bn, nout), lambda i: (i, 0)),
      out_shape=jax.ShapeDtypeStruct((nt, nout), jnp.float32),
  )


def kernel(features, coordinates, W, bias):
  n, nin = features.shape
  nout = W.shape[2]
  npad = -(-n // 2048) * 2048
  pad = npad - n

  coords = coordinates.astype(jnp.int32)
  bcol = jnp.pad(coords[:, 0], (0, pad))
  xcol = jnp.pad(coords[:, 1], (0, pad))
  ycol = jnp.pad(coords[:, 2], (0, pad))
  fext = jnp.concatenate(
      [features, jnp.zeros((8, nin), features.dtype)], axis=0)

  table = _build_table(npad, n, n)(bcol, xcol, ycol)
  tlines, pwords = _tap_lines(npad)(
      table.reshape(TSLOT, 128), bcol, xcol, ycol)
  fix = _extract(npad * 9, 1152)(tlines, pwords.reshape(-1, 1))
  g = _gather_rows(npad, nin)(fix.reshape(-1), fext)

  bn = 512
  nt = -(-n // bn) * bn
  out = _tap_matmul(nt, bn, nin, nout)(g, W, bias.reshape(1, nout))
  return out[:n]


# speedup vs baseline: 10.4396x; 9.3407x over previous
import functools

import jax
import jax.numpy as jnp
from jax import lax
from jax.experimental import pallas as pl
from jax.experimental.pallas import tpu as pltpu
from jax.experimental.pallas import tpu_sc as plsc

S = 512
BATCH = 4
SS = S * S
OFF = S + 1
NKEY = BATCH * SS
DUMMY = NKEY + 2 * S + 16
FTBL = 16 * 66 * 1024
TSLOT = FTBL // 128
TAPS = tuple((dx, dy) for dx in (-1, 0, 1) for dy in (-1, 0, 1))

C1 = 128
C2 = 64
R2 = C2 * 9
RP = 640


def _build_table(npad, n, zrow):
  ns = 16
  chunks = npad // ns // C1
  per_w = FTBL // ns

  mesh = plsc.VectorSubcoreMesh(core_axis_name="c", subcore_axis_name="s",
                                num_cores=1)

  @functools.partial(
      pl.kernel, mesh=mesh,
      out_type=jax.ShapeDtypeStruct((FTBL,), jnp.int32),
      scratch_types=[
          pltpu.VMEM((1024,), jnp.int32),
          pltpu.VMEM((C1,), jnp.int32),
          pltpu.VMEM((C1,), jnp.int32),
          pltpu.VMEM((C1,), jnp.int32),
          pltpu.VMEM((C1,), jnp.int32),
          pltpu.VMEM((C1,), jnp.int32),
          pltpu.SemaphoreType.DMA,
      ],
  )
  def build(b_hbm, x_hbm, y_hbm, table_hbm, cbuf, bb, xb, yb, si, sv, sem):
    wid = lax.axis_index("s")
    zr16 = jnp.full((16,), zrow, jnp.int32)
    for j in range(1024 // 16):
      cbuf[pl.ds(j * 16, 16)] = zr16

    def init_body(r, carry):
      pltpu.sync_copy(cbuf, table_hbm.at[pl.ds(wid * per_w + r * 1024, 1024)])
      return carry

    lax.fori_loop(0, per_w // 1024, init_body, 0)
    plsc.subcore_barrier()

    iota = lax.iota(jnp.int32, 16)

    def chunk_body(ch, carry):
      base = wid * (chunks * C1) + ch * C1
      pltpu.sync_copy(b_hbm.at[pl.ds(base, C1)], bb)
      pltpu.sync_copy(x_hbm.at[pl.ds(base, C1)], xb)
      pltpu.sync_copy(y_hbm.at[pl.ds(base, C1)], yb)
      for j in range(C1 // 16):
        bv = bb[pl.ds(j * 16, 16)]
        xv = xb[pl.ds(j * 16, 16)]
        yv = yb[pl.ds(j * 16, 16)]
        key = bv * SS + xv * S + yv
        rowid = base + j * 16 + iota
        live = rowid < n
        si[pl.ds(j * 16, 16)] = jnp.where(live, key + OFF, DUMMY)
        sv[pl.ds(j * 16, 16)] = jnp.where(live, rowid, zrow)
      pltpu.async_copy(sv, table_hbm.at[si], sem).wait()
      return carry

    lax.fori_loop(0, chunks, chunk_body, 0)

  return build


def _tap_lines(npad):
  info = plsc.get_sparse_core_info()
  nc, ns = info.num_cores, info.num_subcores
  nw = nc * ns
  per_w = npad // nw
  chunks = per_w // C2

  mesh = plsc.VectorSubcoreMesh(core_axis_name="c", subcore_axis_name="s")

  @functools.partial(
      pl.kernel, mesh=mesh,
      out_type=(jax.ShapeDtypeStruct((npad * 9, 128), jnp.int32),
                jax.ShapeDtypeStruct((npad * 9,), jnp.int32)),
      scratch_types=[
          pltpu.VMEM((C2,), jnp.int32),
          pltpu.VMEM((C2,), jnp.int32),
          pltpu.VMEM((C2,), jnp.int32),
          pltpu.VMEM((RP,), jnp.int32),
          pltpu.VMEM((RP,), jnp.int32),
          pltpu.VMEM((RP, 128), jnp.int32),
          pltpu.SemaphoreType.DMA,
      ],
  )
  def lines(table_hbm, b_hbm, x_hbm, y_hbm, lines_hbm, pw_hbm,
            bb, xb, yb, trow, pwb, frows, sem1):
    wid = lax.axis_index("s") * nc + lax.axis_index("c")
    iota16 = lax.iota(jnp.int32, 16)
    zero16 = jnp.zeros((16,), jnp.int32)
    for t in range((RP - R2) // 16):
      trow[pl.ds(R2 + t * 16, 16)] = (DUMMY >> 7) + 1 + t * 16 + iota16
      pwb[pl.ds(R2 + t * 16, 16)] = zero16

    def chunk_body(ch, carry):
      base = wid * per_w + ch * C2
      pltpu.sync_copy(b_hbm.at[pl.ds(base, C2)], bb)
      pltpu.sync_copy(x_hbm.at[pl.ds(base, C2)], xb)
      pltpu.sync_copy(y_hbm.at[pl.ds(base, C2)], yb)
      for j in range(C2 // 16):
        bv = bb[pl.ds(j * 16, 16)]
        xv = xb[pl.ds(j * 16, 16)]
        yv = yb[pl.ds(j * 16, 16)]
        key = bv * SS + xv * S + yv + OFF
        for k, (dx, dy) in enumerate(TAPS):
          nk = key + (dx * S + dy)
          conds = []
          if dx < 0:
            conds.append(xv > 0)
          if dx > 0:
            conds.append(xv < S - 1)
          if dy < 0:
            conds.append(yv > 0)
          if dy > 0:
            conds.append(yv < S - 1)
          if conds:
            ok = conds[0]
            for c in conds[1:]:
              ok = ok & c
            nk = jnp.where(ok, nk, DUMMY)
          trow[pl.ds(k * C2 + j * 16, 16)] = nk >> 7
          pwb[pl.ds(k * C2 + j * 16, 16)] = nk & 127
      hs = [pltpu.async_copy(table_hbm.at[trow.at[pl.ds(q * 128, 128)]],
                             frows.at[pl.ds(q * 128, 128)], sem1)
            for q in range(RP // 128)]
      for h in hs:
        h.wait()
      pltpu.sync_copy(frows.at[pl.ds(0, R2)],
                      lines_hbm.at[pl.ds(base * 9, R2)])
      pltpu.sync_copy(pwb.at[pl.ds(0, R2)], pw_hbm.at[pl.ds(base * 9, R2)])
      return carry

    lax.fori_loop(0, chunks, chunk_body, 0)

  return lines


def _extract(nrows, rb, zrow):
  def body(l_ref, p_ref, o_ref):
    sel = lax.broadcasted_iota(jnp.int32, (rb, 128), 1) == p_ref[...]
    val = jnp.sum(jnp.where(sel, l_ref[...], 0), axis=1, keepdims=True)
    slot = (pl.program_id(0) * rb
            + lax.broadcasted_iota(jnp.int32, (rb, 1), 0)) % 576
    o_ref[...] = jnp.where(val == zrow, zrow + slot, val)

  return pl.pallas_call(
      body,
      grid=(nrows // rb,),
      in_specs=[
          pl.BlockSpec((rb, 128), lambda i: (i, 0)),
          pl.BlockSpec((rb, 1), lambda i: (i, 0)),
      ],
      out_specs=pl.BlockSpec((rb, 1), lambda i: (i, 0)),
      out_shape=jax.ShapeDtypeStruct((nrows, 1), jnp.int32),
  )


def _gather_rows(npad, nin, zrow):
  info = plsc.get_sparse_core_info()
  nc, ns = info.num_cores, info.num_subcores
  nw = nc * ns
  per_w = npad // nw
  chunks = per_w // C2

  mesh = plsc.VectorSubcoreMesh(core_axis_name="c", subcore_axis_name="s")

  @functools.partial(
      pl.kernel, mesh=mesh,
      out_type=jax.ShapeDtypeStruct((9, npad, nin), jnp.float32),
      scratch_types=[
          pltpu.VMEM((RP,), jnp.int32),
          pltpu.VMEM((RP, nin), jnp.float32),
          pltpu.SemaphoreType.DMA,
      ],
  )
  def gather(fix_hbm, f_hbm, g_hbm, fixb, gbuf, sem2):
    wid = lax.axis_index("s") * nc + lax.axis_index("c")
    iota16 = lax.iota(jnp.int32, 16)
    for t in range((RP - R2) // 16):
      fixb[pl.ds(R2 + t * 16, 16)] = zrow + 576 + t * 16 + iota16

    def chunk_body(ch, carry):
      base = wid * per_w + ch * C2
      pltpu.sync_copy(fix_hbm.at[pl.ds(base * 9, R2)], fixb.at[pl.ds(0, R2)])
      hs = [pltpu.async_copy(f_hbm.at[fixb.at[pl.ds(q * 128, 128)]],
                             gbuf.at[pl.ds(q * 128, 128)], sem2)
            for q in range(RP // 128)]
      for h in hs:
        h.wait()
      for k in range(9):
        pltpu.sync_copy(gbuf.at[pl.ds(k * C2, C2)],
                        g_hbm.at[k, pl.ds(base, C2)])
      return carry

    lax.fori_loop(0, chunks, chunk_body, 0)

  return gather


def _tap_matmul(nt, bn, nin, nout):
  def body(g_ref, w_ref, b_ref, o_ref):
    acc = b_ref[...].astype(jnp.float32)
    for k in range(9):
      acc = acc + jnp.dot(g_ref[k], w_ref[k],
                          preferred_element_type=jnp.float32)
    o_ref[...] = jnp.maximum(acc, 0.0)

  return pl.pallas_call(
      body,
      grid=(nt // bn,),
      in_specs=[
          pl.BlockSpec((9, bn, nin), lambda i: (0, i, 0)),
          pl.BlockSpec((9, nin, nout), lambda i: (0, 0, 0)),
          pl.BlockSpec((1, nout), lambda i: (0, 0)),
      ],
      out_specs=pl.BlockSpec((bn, nout), lambda i: (i, 0)),
      out_shape=jax.ShapeDtypeStruct((nt, nout), jnp.float32),
  )


def kernel(features, coordinates, W, bias):
  n, nin = features.shape
  nout = W.shape[2]
  npad = -(-n // 2048) * 2048
  pad = npad - n

  coords = coordinates.astype(jnp.int32)
  bcol = jnp.pad(coords[:, 0], (0, pad))
  xcol = jnp.pad(coords[:, 1], (0, pad))
  ycol = jnp.pad(coords[:, 2], (0, pad))
  fext = jnp.concatenate(
      [features, jnp.zeros((648, nin), features.dtype)], axis=0)

  table = _build_table(npad, n, n)(bcol, xcol, ycol)
  tlines, pwords = _tap_lines(npad)(
      table.reshape(TSLOT, 128), bcol, xcol, ycol)
  fix = _extract(npad * 9, 1152, n)(tlines, pwords.reshape(-1, 1))
  g = _gather_rows(npad, nin, n)(fix.reshape(-1), fext)

  bn = 512
  nt = -(-n // bn) * bn
  out = _tap_matmul(nt, bn, nin, nout)(g, W, bias.reshape(1, nout))
  return out[:n]


# speedup vs baseline: 11.6609x; 1.1170x over previous
import functools

import jax
import jax.numpy as jnp
from jax import lax
from jax.experimental import pallas as pl
from jax.experimental.pallas import tpu as pltpu
from jax.experimental.pallas import tpu_sc as plsc

S = 512
BATCH = 4
SS = S * S
OFF = S + 1
NKEY = BATCH * SS
DUMMY = NKEY + 2 * S + 16
FTBL = 16 * 66 * 1024
TSLOT = FTBL // 128
TAPS = tuple((dx, dy) for dx in (-1, 0, 1) for dy in (-1, 0, 1))

C1 = 128
C2 = 64
R2 = C2 * 9
RP = 640


def _build_table(npad, n, zrow):
  ns = 16
  chunks = npad // ns // C1
  per_w = FTBL // ns

  mesh = plsc.VectorSubcoreMesh(core_axis_name="c", subcore_axis_name="s",
                                num_cores=1)

  @functools.partial(
      pl.kernel, mesh=mesh,
      out_type=jax.ShapeDtypeStruct((FTBL,), jnp.int32),
      scratch_types=[
          pltpu.VMEM((1024,), jnp.int32),
          pltpu.VMEM((C1,), jnp.int32),
          pltpu.VMEM((C1,), jnp.int32),
          pltpu.VMEM((C1,), jnp.int32),
          pltpu.VMEM((C1,), jnp.int32),
          pltpu.VMEM((C1,), jnp.int32),
          pltpu.SemaphoreType.DMA,
      ],
  )
  def build(b_hbm, x_hbm, y_hbm, table_hbm, cbuf, bb, xb, yb, si, sv, sem):
    wid = lax.axis_index("s")
    zr16 = jnp.full((16,), zrow, jnp.int32)
    for j in range(1024 // 16):
      cbuf[pl.ds(j * 16, 16)] = zr16

    def init_body(r, carry):
      pltpu.sync_copy(cbuf, table_hbm.at[pl.ds(wid * per_w + r * 1024, 1024)])
      return carry

    lax.fori_loop(0, per_w // 1024, init_body, 0)
    plsc.subcore_barrier()

    iota = lax.iota(jnp.int32, 16)

    def chunk_body(ch, carry):
      base = wid * (chunks * C1) + ch * C1
      pltpu.sync_copy(b_hbm.at[pl.ds(base, C1)], bb)
      pltpu.sync_copy(x_hbm.at[pl.ds(base, C1)], xb)
      pltpu.sync_copy(y_hbm.at[pl.ds(base, C1)], yb)
      for j in range(C1 // 16):
        bv = bb[pl.ds(j * 16, 16)]
        xv = xb[pl.ds(j * 16, 16)]
        yv = yb[pl.ds(j * 16, 16)]
        key = bv * SS + xv * S + yv
        rowid = base + j * 16 + iota
        live = rowid < n
        si[pl.ds(j * 16, 16)] = jnp.where(live, key + OFF, DUMMY)
        sv[pl.ds(j * 16, 16)] = jnp.where(live, rowid, zrow)
      pltpu.async_copy(sv, table_hbm.at[si], sem).wait()
      return carry

    lax.fori_loop(0, chunks, chunk_body, 0)

  return build


def _tap_lines(npad):
  info = plsc.get_sparse_core_info()
  nc, ns = info.num_cores, info.num_subcores
  nw = nc * ns
  per_w = npad // nw
  chunks = per_w // C2

  mesh = plsc.VectorSubcoreMesh(core_axis_name="c", subcore_axis_name="s")

  @functools.partial(
      pl.kernel, mesh=mesh,
      out_type=(jax.ShapeDtypeStruct((npad * 9, 128), jnp.int32),
                jax.ShapeDtypeStruct((npad * 9,), jnp.int32)),
      scratch_types=[
          pltpu.VMEM((C2,), jnp.int32),
          pltpu.VMEM((C2,), jnp.int32),
          pltpu.VMEM((C2,), jnp.int32),
          pltpu.VMEM((RP,), jnp.int32),
          pltpu.VMEM((RP,), jnp.int32),
          pltpu.VMEM((RP, 128), jnp.int32),
          pltpu.SemaphoreType.DMA,
      ],
  )
  def lines(table_hbm, b_hbm, x_hbm, y_hbm, lines_hbm, pw_hbm,
            bb, xb, yb, trow, pwb, frows, sem1):
    wid = lax.axis_index("s") * nc + lax.axis_index("c")
    iota16 = lax.iota(jnp.int32, 16)
    zero16 = jnp.zeros((16,), jnp.int32)
    for t in range((RP - R2) // 16):
      trow[pl.ds(R2 + t * 16, 16)] = (DUMMY >> 7) + 1 + t * 16 + iota16
      pwb[pl.ds(R2 + t * 16, 16)] = zero16

    def chunk_body(ch, carry):
      base = wid * per_w + ch * C2
      pltpu.sync_copy(b_hbm.at[pl.ds(base, C2)], bb)
      pltpu.sync_copy(x_hbm.at[pl.ds(base, C2)], xb)
      pltpu.sync_copy(y_hbm.at[pl.ds(base, C2)], yb)
      for j in range(C2 // 16):
        bv = bb[pl.ds(j * 16, 16)]
        xv = xb[pl.ds(j * 16, 16)]
        yv = yb[pl.ds(j * 16, 16)]
        key = bv * SS + xv * S + yv + OFF
        for k, (dx, dy) in enumerate(TAPS):
          nk = key + (dx * S + dy)
          conds = []
          if dx < 0:
            conds.append(xv > 0)
          if dx > 0:
            conds.append(xv < S - 1)
          if dy < 0:
            conds.append(yv > 0)
          if dy > 0:
            conds.append(yv < S - 1)
          if conds:
            ok = conds[0]
            for c in conds[1:]:
              ok = ok & c
            nk = jnp.where(ok, nk, DUMMY)
          trow[pl.ds(k * C2 + j * 16, 16)] = nk >> 7
          pwb[pl.ds(k * C2 + j * 16, 16)] = nk & 127
      hs = [pltpu.async_copy(table_hbm.at[trow.at[pl.ds(q * 128, 128)]],
                             frows.at[pl.ds(q * 128, 128)], sem1)
            for q in range(RP // 128)]
      for h in hs:
        h.wait()
      pltpu.sync_copy(frows.at[pl.ds(0, R2)],
                      lines_hbm.at[pl.ds(base * 9, R2)])
      pltpu.sync_copy(pwb.at[pl.ds(0, R2)], pw_hbm.at[pl.ds(base * 9, R2)])
      return carry

    lax.fori_loop(0, chunks, chunk_body, 0)

  return lines


def _extract(nrows, rb, zrow):
  def body(l_ref, p_ref, o_ref):
    sel = lax.broadcasted_iota(jnp.int32, (rb, 128), 1) == p_ref[...]
    val = jnp.sum(jnp.where(sel, l_ref[...], 0), axis=1, keepdims=True)
    slot = (pl.program_id(0) * rb
            + lax.broadcasted_iota(jnp.int32, (rb, 1), 0)) % 576
    o_ref[...] = jnp.where(val == zrow, zrow + slot, val)

  return pl.pallas_call(
      body,
      grid=(nrows // rb,),
      in_specs=[
          pl.BlockSpec((rb, 128), lambda i: (i, 0)),
          pl.BlockSpec((rb, 1), lambda i: (i, 0)),
      ],
      out_specs=pl.BlockSpec((rb, 1), lambda i: (i, 0)),
      out_shape=jax.ShapeDtypeStruct((nrows, 1), jnp.int32),
  )


def _gather_rows(npad, nin, zrow):
  info = plsc.get_sparse_core_info()
  nc, ns = info.num_cores, info.num_subcores
  nw = nc * ns
  per_w = npad // nw
  chunks = per_w // C2

  mesh = plsc.VectorSubcoreMesh(core_axis_name="c", subcore_axis_name="s")

  @functools.partial(
      pl.kernel, mesh=mesh,
      out_type=jax.ShapeDtypeStruct((9, npad, nin), jnp.float32),
      scratch_types=[
          pltpu.VMEM((RP,), jnp.int32),
          pltpu.VMEM((RP, nin), jnp.float32),
          pltpu.SemaphoreType.DMA,
      ],
  )
  def gather(fix_hbm, f_hbm, g_hbm, fixb, gbuf, sem2):
    wid = lax.axis_index("s") * nc + lax.axis_index("c")
    iota16 = lax.iota(jnp.int32, 16)
    for t in range((RP - R2) // 16):
      fixb[pl.ds(R2 + t * 16, 16)] = zrow + 576 + t * 16 + iota16

    def chunk_body(ch, carry):
      base = wid * per_w + ch * C2
      pltpu.sync_copy(fix_hbm.at[pl.ds(base * 9, R2)], fixb.at[pl.ds(0, R2)])
      hs = [pltpu.async_copy(f_hbm.at[fixb.at[pl.ds(q * 128, 128)]],
                             gbuf.at[pl.ds(q * 128, 128)], sem2)
            for q in range(RP // 128)]
      for h in hs:
        h.wait()
      for k in range(9):
        pltpu.sync_copy(gbuf.at[pl.ds(k * C2, C2)],
                        g_hbm.at[k, pl.ds(base, C2)])
      return carry

    lax.fori_loop(0, chunks, chunk_body, 0)

  return gather


def _tap_matmul(nt, bn, nin, nout):
  def body(g_ref, w_ref, b_ref, o_ref):
    acc = b_ref[...].astype(jnp.float32)
    for k in range(9):
      acc = acc + jnp.dot(g_ref[k], w_ref[k],
                          preferred_element_type=jnp.float32)
    o_ref[...] = jnp.maximum(acc, 0.0)

  return pl.pallas_call(
      body,
      grid=(nt // bn,),
      in_specs=[
          pl.BlockSpec((9, bn, nin), lambda i: (0, i, 0)),
          pl.BlockSpec((9, nin, nout), lambda i: (0, 0, 0)),
          pl.BlockSpec((1, nout), lambda i: (0, 0)),
      ],
      out_specs=pl.BlockSpec((bn, nout), lambda i: (i, 0)),
      out_shape=jax.ShapeDtypeStruct((nt, nout), jnp.float32),
  )


def kernel(features, coordinates, W, bias):
  n, nin = features.shape
  nout = W.shape[2]
  npad = -(-n // 2048) * 2048
  pad = npad - n

  coords = coordinates.astype(jnp.int32)
  bcol = jnp.pad(coords[:, 0], (0, pad))
  xcol = jnp.pad(coords[:, 1], (0, pad))
  ycol = jnp.pad(coords[:, 2], (0, pad))
  fext = jnp.concatenate(
      [features, jnp.zeros((648, nin), features.dtype)], axis=0)

  table = _build_table(npad, n, n)(bcol, xcol, ycol)
  tlines, pwords = _tap_lines(npad)(
      table.reshape(TSLOT, 128), bcol, xcol, ycol)
  fix = _extract(npad * 9, 4608, n)(tlines, pwords.reshape(-1, 1))
  g = _gather_rows(npad, nin, n)(fix.reshape(-1), fext)

  bn = 1024
  nt = -(-n // bn) * bn
  out = _tap_matmul(nt, bn, nin, nout)(g, W, bias.reshape(1, nout))
  return out[:n]


# speedup vs baseline: 11.8559x; 1.0167x over previous
import functools

import jax
import jax.numpy as jnp
from jax import lax
from jax.experimental import pallas as pl
from jax.experimental.pallas import tpu as pltpu
from jax.experimental.pallas import tpu_sc as plsc

S = 512
BATCH = 4
SS = S * S
OFF = S + 1
NKEY = BATCH * SS
DUMMY = NKEY + 2 * S + 16
FTBL = 16 * 66 * 1024
TSLOT = FTBL // 128
TAPS = tuple((dx, dy) for dx in (-1, 0, 1) for dy in (-1, 0, 1))

C1 = 128
C2 = 64
R2 = C2 * 9
RP = 640


def _build_table(npad, n, zrow):
  ns = 16
  chunks = npad // ns // C1
  per_w = FTBL // ns

  mesh = plsc.VectorSubcoreMesh(core_axis_name="c", subcore_axis_name="s",
                                num_cores=1)

  @functools.partial(
      pl.kernel, mesh=mesh,
      out_type=jax.ShapeDtypeStruct((FTBL,), jnp.int32),
      scratch_types=[
          pltpu.VMEM((1024,), jnp.int32),
          pltpu.VMEM((C1,), jnp.int32),
          pltpu.VMEM((C1,), jnp.int32),
          pltpu.VMEM((C1,), jnp.int32),
          pltpu.VMEM((C1,), jnp.int32),
          pltpu.VMEM((C1,), jnp.int32),
          pltpu.SemaphoreType.DMA,
      ],
  )
  def build(b_hbm, x_hbm, y_hbm, table_hbm, cbuf, bb, xb, yb, si, sv, sem):
    wid = lax.axis_index("s")
    zr16 = jnp.full((16,), zrow, jnp.int32)
    for j in range(1024 // 16):
      cbuf[pl.ds(j * 16, 16)] = zr16

    def init_body(r, carry):
      pltpu.sync_copy(cbuf, table_hbm.at[pl.ds(wid * per_w + r * 1024, 1024)])
      return carry

    lax.fori_loop(0, per_w // 1024, init_body, 0)
    plsc.subcore_barrier()

    iota = lax.iota(jnp.int32, 16)

    def chunk_body(ch, carry):
      base = wid * (chunks * C1) + ch * C1
      pltpu.sync_copy(b_hbm.at[pl.ds(base, C1)], bb)
      pltpu.sync_copy(x_hbm.at[pl.ds(base, C1)], xb)
      pltpu.sync_copy(y_hbm.at[pl.ds(base, C1)], yb)
      for j in range(C1 // 16):
        bv = bb[pl.ds(j * 16, 16)]
        xv = xb[pl.ds(j * 16, 16)]
        yv = yb[pl.ds(j * 16, 16)]
        key = bv * SS + xv * S + yv
        rowid = base + j * 16 + iota
        live = rowid < n
        si[pl.ds(j * 16, 16)] = jnp.where(live, key + OFF, DUMMY)
        sv[pl.ds(j * 16, 16)] = jnp.where(live, rowid, zrow)
      pltpu.async_copy(sv, table_hbm.at[si], sem).wait()
      return carry

    lax.fori_loop(0, chunks, chunk_body, 0)

  return build


def _tap_lines(npad):
  info = plsc.get_sparse_core_info()
  nc, ns = info.num_cores, info.num_subcores
  nw = nc * ns
  per_w = npad // nw
  chunks = per_w // C2

  mesh = plsc.VectorSubcoreMesh(core_axis_name="c", subcore_axis_name="s")

  @functools.partial(
      pl.kernel, mesh=mesh,
      out_type=(jax.ShapeDtypeStruct((npad * 9, 128), jnp.int32),
                jax.ShapeDtypeStruct((npad * 9,), jnp.int32)),
      scratch_types=[
          pltpu.VMEM((C2,), jnp.int32),
          pltpu.VMEM((C2,), jnp.int32),
          pltpu.VMEM((C2,), jnp.int32),
          pltpu.VMEM((RP,), jnp.int32),
          pltpu.VMEM((RP,), jnp.int32),
          pltpu.VMEM((RP, 128), jnp.int32),
          pltpu.SemaphoreType.DMA,
      ],
  )
  def lines(table_hbm, b_hbm, x_hbm, y_hbm, lines_hbm, pw_hbm,
            bb, xb, yb, trow, pwb, frows, sem1):
    wid = lax.axis_index("s") * nc + lax.axis_index("c")
    iota16 = lax.iota(jnp.int32, 16)
    zero16 = jnp.zeros((16,), jnp.int32)
    for t in range((RP - R2) // 16):
      trow[pl.ds(R2 + t * 16, 16)] = (DUMMY >> 7) + 1 + t * 16 + iota16
      pwb[pl.ds(R2 + t * 16, 16)] = zero16

    def chunk_body(ch, carry):
      base = wid * per_w + ch * C2
      pltpu.sync_copy(b_hbm.at[pl.ds(base, C2)], bb)
      pltpu.sync_copy(x_hbm.at[pl.ds(base, C2)], xb)
      pltpu.sync_copy(y_hbm.at[pl.ds(base, C2)], yb)
      for j in range(C2 // 16):
        bv = bb[pl.ds(j * 16, 16)]
        xv = xb[pl.ds(j * 16, 16)]
        yv = yb[pl.ds(j * 16, 16)]
        key = bv * SS + xv * S + yv + OFF
        for k, (dx, dy) in enumerate(TAPS):
          nk = key + (dx * S + dy)
          conds = []
          if dx < 0:
            conds.append(xv > 0)
          if dx > 0:
            conds.append(xv < S - 1)
          if dy < 0:
            conds.append(yv > 0)
          if dy > 0:
            conds.append(yv < S - 1)
          if conds:
            ok = conds[0]
            for c in conds[1:]:
              ok = ok & c
            nk = jnp.where(ok, nk, DUMMY)
          trow[pl.ds(k * C2 + j * 16, 16)] = nk >> 7
          pwb[pl.ds(k * C2 + j * 16, 16)] = nk & 127
      hs = [pltpu.async_copy(table_hbm.at[trow.at[pl.ds(q * 128, 128)]],
                             frows.at[pl.ds(q * 128, 128)], sem1)
            for q in range(RP // 128)]
      for h in hs:
        h.wait()
      pltpu.sync_copy(frows.at[pl.ds(0, R2)],
                      lines_hbm.at[pl.ds(base * 9, R2)])
      pltpu.sync_copy(pwb.at[pl.ds(0, R2)], pw_hbm.at[pl.ds(base * 9, R2)])
      return carry

    lax.fori_loop(0, chunks, chunk_body, 0)

  return lines


def _extract(nrows, rb, zrow):
  def body(l_ref, p_ref, o_ref):
    sel = lax.broadcasted_iota(jnp.int32, (rb, 128), 1) == p_ref[...]
    val = jnp.sum(jnp.where(sel, l_ref[...], 0), axis=1, keepdims=True)
    slot = (pl.program_id(0) * rb
            + lax.broadcasted_iota(jnp.int32, (rb, 1), 0)) % 576
    o_ref[...] = jnp.where(val == zrow, zrow + slot, val)

  return pl.pallas_call(
      body,
      grid=(nrows // rb,),
      in_specs=[
          pl.BlockSpec((rb, 128), lambda i: (i, 0)),
          pl.BlockSpec((rb, 1), lambda i: (i, 0)),
      ],
      out_specs=pl.BlockSpec((rb, 1), lambda i: (i, 0)),
      out_shape=jax.ShapeDtypeStruct((nrows, 1), jnp.int32),
  )


def _gather_rows(npad, nin, zrow):
  info = plsc.get_sparse_core_info()
  nc, ns = info.num_cores, info.num_subcores
  nw = nc * ns
  per_w = npad // nw
  chunks = per_w // C2

  mesh = plsc.VectorSubcoreMesh(core_axis_name="c", subcore_axis_name="s")

  @functools.partial(
      pl.kernel, mesh=mesh,
      out_type=jax.ShapeDtypeStruct((9, npad, nin), jnp.float32),
      scratch_types=[
          pltpu.VMEM((RP,), jnp.int32),
          pltpu.VMEM((RP, nin), jnp.float32),
          pltpu.SemaphoreType.DMA,
      ],
  )
  def gather(fix_hbm, f_hbm, g_hbm, fixb, gbuf, sem2):
    wid = lax.axis_index("s") * nc + lax.axis_index("c")
    iota16 = lax.iota(jnp.int32, 16)
    for t in range((RP - R2) // 16):
      fixb[pl.ds(R2 + t * 16, 16)] = zrow + 576 + t * 16 + iota16

    def chunk_body(ch, carry):
      base = wid * per_w + ch * C2
      pltpu.sync_copy(fix_hbm.at[pl.ds(base * 9, R2)], fixb.at[pl.ds(0, R2)])
      hs = [pltpu.async_copy(f_hbm.at[fixb.at[pl.ds(q * 128, 128)]],
                             gbuf.at[pl.ds(q * 128, 128)], sem2)
            for q in range(RP // 128)]
      for h in hs:
        h.wait()
      for k in range(9):
        pltpu.sync_copy(gbuf.at[pl.ds(k * C2, C2)],
                        g_hbm.at[k, pl.ds(base, C2)])
      return carry

    lax.fori_loop(0, chunks, chunk_body, 0)

  return gather


def _tap_matmul(nt, bn, nin, nout):
  def body(g_ref, w_ref, b_ref, o_ref):
    acc = b_ref[...].astype(jnp.float32)
    for k in range(9):
      acc = acc + jnp.dot(g_ref[k], w_ref[k],
                          preferred_element_type=jnp.float32)
    o_ref[...] = jnp.maximum(acc, 0.0)

  return pl.pallas_call(
      body,
      grid=(nt // bn,),
      in_specs=[
          pl.BlockSpec((9, bn, nin), lambda i: (0, i, 0)),
          pl.BlockSpec((9, nin, nout), lambda i: (0, 0, 0)),
          pl.BlockSpec((1, nout), lambda i: (0, 0)),
      ],
      out_specs=pl.BlockSpec((bn, nout), lambda i: (i, 0)),
      out_shape=jax.ShapeDtypeStruct((nt, nout), jnp.float32),
  )


def kernel(features, coordinates, W, bias):
  n, nin = features.shape
  nout = W.shape[2]
  npad = -(-n // 2048) * 2048
  pad = npad - n

  coords = coordinates.astype(jnp.int32)
  bcol = jnp.pad(coords[:, 0], (0, pad))
  xcol = jnp.pad(coords[:, 1], (0, pad))
  ycol = jnp.pad(coords[:, 2], (0, pad))
  fext = jnp.concatenate(
      [features, jnp.zeros((648, nin), features.dtype)], axis=0)

  table = _build_table(npad, n, n)(bcol, xcol, ycol)
  tlines, pwords = _tap_lines(npad)(
      table.reshape(TSLOT, 128), bcol, xcol, ycol)
  fix = _extract(npad * 9, 9216, n)(tlines, pwords.reshape(-1, 1))
  g = _gather_rows(npad, nin, n)(fix.reshape(-1), fext)

  bn = 2048
  nt = -(-n // bn) * bn
  out = _tap_matmul(nt, bn, nin, nout)(g, W, bias.reshape(1, nout))
  return out[:n]


# speedup vs baseline: 12.0238x; 1.0142x over previous
import functools

import jax
import jax.numpy as jnp
from jax import lax
from jax.experimental import pallas as pl
from jax.experimental.pallas import tpu as pltpu
from jax.experimental.pallas import tpu_sc as plsc

S = 512
BATCH = 4
SS = S * S
OFF = S + 1
NKEY = BATCH * SS
DUMMY = NKEY + 2 * S + 16
FTBL = 16 * 66 * 1024
TSLOT = FTBL // 128
TAPS = tuple((dx, dy) for dx in (-1, 0, 1) for dy in (-1, 0, 1))

C1 = 128
C2 = 80
R2 = C2 * 9
RP = 768


def _build_table(npad, n, zrow):
  ns = 16
  chunks = npad // ns // C1
  per_w = FTBL // ns

  mesh = plsc.VectorSubcoreMesh(core_axis_name="c", subcore_axis_name="s",
                                num_cores=1)

  @functools.partial(
      pl.kernel, mesh=mesh,
      out_type=jax.ShapeDtypeStruct((FTBL,), jnp.int32),
      scratch_types=[
          pltpu.VMEM((1024,), jnp.int32),
          pltpu.VMEM((C1,), jnp.int32),
          pltpu.VMEM((C1,), jnp.int32),
          pltpu.VMEM((C1,), jnp.int32),
          pltpu.VMEM((C1,), jnp.int32),
          pltpu.VMEM((C1,), jnp.int32),
          pltpu.SemaphoreType.DMA,
      ],
  )
  def build(b_hbm, x_hbm, y_hbm, table_hbm, cbuf, bb, xb, yb, si, sv, sem):
    wid = lax.axis_index("s")
    zr16 = jnp.full((16,), zrow, jnp.int32)
    for j in range(1024 // 16):
      cbuf[pl.ds(j * 16, 16)] = zr16

    def init_body(r, carry):
      pltpu.sync_copy(cbuf, table_hbm.at[pl.ds(wid * per_w + r * 1024, 1024)])
      return carry

    lax.fori_loop(0, per_w // 1024, init_body, 0)
    plsc.subcore_barrier()

    iota = lax.iota(jnp.int32, 16)

    def chunk_body(ch, carry):
      base = wid * (chunks * C1) + ch * C1
      pltpu.sync_copy(b_hbm.at[pl.ds(base, C1)], bb)
      pltpu.sync_copy(x_hbm.at[pl.ds(base, C1)], xb)
      pltpu.sync_copy(y_hbm.at[pl.ds(base, C1)], yb)
      for j in range(C1 // 16):
        bv = bb[pl.ds(j * 16, 16)]
        xv = xb[pl.ds(j * 16, 16)]
        yv = yb[pl.ds(j * 16, 16)]
        key = bv * SS + xv * S + yv
        rowid = base + j * 16 + iota
        live = rowid < n
        si[pl.ds(j * 16, 16)] = jnp.where(live, key + OFF, DUMMY)
        sv[pl.ds(j * 16, 16)] = jnp.where(live, rowid, zrow)
      pltpu.async_copy(sv, table_hbm.at[si], sem).wait()
      return carry

    lax.fori_loop(0, chunks, chunk_body, 0)

  return build


def _tap_lines(npad):
  info = plsc.get_sparse_core_info()
  nc, ns = info.num_cores, info.num_subcores
  nw = nc * ns
  per_w = npad // nw
  chunks = per_w // C2

  mesh = plsc.VectorSubcoreMesh(core_axis_name="c", subcore_axis_name="s")

  @functools.partial(
      pl.kernel, mesh=mesh,
      out_type=(jax.ShapeDtypeStruct((npad * 9, 128), jnp.int32),
                jax.ShapeDtypeStruct((npad * 9,), jnp.int32)),
      scratch_types=[
          pltpu.VMEM((C2,), jnp.int32),
          pltpu.VMEM((C2,), jnp.int32),
          pltpu.VMEM((C2,), jnp.int32),
          pltpu.VMEM((RP,), jnp.int32),
          pltpu.VMEM((RP,), jnp.int32),
          pltpu.VMEM((RP, 128), jnp.int32),
          pltpu.SemaphoreType.DMA,
      ],
  )
  def lines(table_hbm, b_hbm, x_hbm, y_hbm, lines_hbm, pw_hbm,
            bb, xb, yb, trow, pwb, frows, sem1):
    wid = lax.axis_index("s") * nc + lax.axis_index("c")
    iota16 = lax.iota(jnp.int32, 16)
    zero16 = jnp.zeros((16,), jnp.int32)
    for t in range((RP - R2) // 16):
      trow[pl.ds(R2 + t * 16, 16)] = (DUMMY >> 7) + 1 + t * 16 + iota16
      pwb[pl.ds(R2 + t * 16, 16)] = zero16

    def chunk_body(ch, carry):
      base = wid * per_w + ch * C2
      pltpu.sync_copy(b_hbm.at[pl.ds(base, C2)], bb)
      pltpu.sync_copy(x_hbm.at[pl.ds(base, C2)], xb)
      pltpu.sync_copy(y_hbm.at[pl.ds(base, C2)], yb)
      for j in range(C2 // 16):
        bv = bb[pl.ds(j * 16, 16)]
        xv = xb[pl.ds(j * 16, 16)]
        yv = yb[pl.ds(j * 16, 16)]
        key = bv * SS + xv * S + yv + OFF
        for k, (dx, dy) in enumerate(TAPS):
          nk = key + (dx * S + dy)
          conds = []
          if dx < 0:
            conds.append(xv > 0)
          if dx > 0:
            conds.append(xv < S - 1)
          if dy < 0:
            conds.append(yv > 0)
          if dy > 0:
            conds.append(yv < S - 1)
          if conds:
            ok = conds[0]
            for c in conds[1:]:
              ok = ok & c
            nk = jnp.where(ok, nk, DUMMY)
          trow[pl.ds(k * C2 + j * 16, 16)] = nk >> 7
          pwb[pl.ds(k * C2 + j * 16, 16)] = nk & 127
      hs = [pltpu.async_copy(table_hbm.at[trow.at[pl.ds(q * 128, 128)]],
                             frows.at[pl.ds(q * 128, 128)], sem1)
            for q in range(RP // 128)]
      for h in hs:
        h.wait()
      pltpu.sync_copy(frows.at[pl.ds(0, R2)],
                      lines_hbm.at[pl.ds(base * 9, R2)])
      pltpu.sync_copy(pwb.at[pl.ds(0, R2)], pw_hbm.at[pl.ds(base * 9, R2)])
      return carry

    lax.fori_loop(0, chunks, chunk_body, 0)

  return lines


def _extract(nrows, rb, zrow):
  def body(l_ref, p_ref, o_ref):
    sel = lax.broadcasted_iota(jnp.int32, (rb, 128), 1) == p_ref[...]
    val = jnp.sum(jnp.where(sel, l_ref[...], 0), axis=1, keepdims=True)
    slot = (pl.program_id(0) * rb
            + lax.broadcasted_iota(jnp.int32, (rb, 1), 0)) % 576
    o_ref[...] = jnp.where(val == zrow, zrow + slot, val)

  return pl.pallas_call(
      body,
      grid=(nrows // rb,),
      in_specs=[
          pl.BlockSpec((rb, 128), lambda i: (i, 0)),
          pl.BlockSpec((rb, 1), lambda i: (i, 0)),
      ],
      out_specs=pl.BlockSpec((rb, 1), lambda i: (i, 0)),
      out_shape=jax.ShapeDtypeStruct((nrows, 1), jnp.int32),
  )


def _gather_rows(npad, nin, zrow):
  info = plsc.get_sparse_core_info()
  nc, ns = info.num_cores, info.num_subcores
  nw = nc * ns
  per_w = npad // nw
  chunks = per_w // C2

  mesh = plsc.VectorSubcoreMesh(core_axis_name="c", subcore_axis_name="s")

  @functools.partial(
      pl.kernel, mesh=mesh,
      out_type=jax.ShapeDtypeStruct((9, npad, nin), jnp.float32),
      scratch_types=[
          pltpu.VMEM((RP,), jnp.int32),
          pltpu.VMEM((RP, nin), jnp.float32),
          pltpu.SemaphoreType.DMA,
      ],
  )
  def gather(fix_hbm, f_hbm, g_hbm, fixb, gbuf, sem2):
    wid = lax.axis_index("s") * nc + lax.axis_index("c")
    iota16 = lax.iota(jnp.int32, 16)
    for t in range((RP - R2) // 16):
      fixb[pl.ds(R2 + t * 16, 16)] = zrow + 576 + t * 16 + iota16

    def chunk_body(ch, carry):
      base = wid * per_w + ch * C2
      pltpu.sync_copy(fix_hbm.at[pl.ds(base * 9, R2)], fixb.at[pl.ds(0, R2)])
      hs = [pltpu.async_copy(f_hbm.at[fixb.at[pl.ds(q * 128, 128)]],
                             gbuf.at[pl.ds(q * 128, 128)], sem2)
            for q in range(RP // 128)]
      for h in hs:
        h.wait()
      for k in range(9):
        pltpu.sync_copy(gbuf.at[pl.ds(k * C2, C2)],
                        g_hbm.at[k, pl.ds(base, C2)])
      return carry

    lax.fori_loop(0, chunks, chunk_body, 0)

  return gather


def _tap_matmul(nt, bn, nin, nout):
  def body(g_ref, w_ref, b_ref, o_ref):
    acc = b_ref[...].astype(jnp.float32)
    for k in range(9):
      acc = acc + jnp.dot(g_ref[k], w_ref[k],
                          preferred_element_type=jnp.float32)
    o_ref[...] = jnp.maximum(acc, 0.0)

  return pl.pallas_call(
      body,
      grid=(nt // bn,),
      in_specs=[
          pl.BlockSpec((9, bn, nin), lambda i: (0, i, 0)),
          pl.BlockSpec((9, nin, nout), lambda i: (0, 0, 0)),
          pl.BlockSpec((1, nout), lambda i: (0, 0)),
      ],
      out_specs=pl.BlockSpec((bn, nout), lambda i: (i, 0)),
      out_shape=jax.ShapeDtypeStruct((nt, nout), jnp.float32),
  )


def kernel(features, coordinates, W, bias):
  n, nin = features.shape
  nout = W.shape[2]
  npad = -(-n // 2048) * 2048
  pad = npad - n

  coords = coordinates.astype(jnp.int32)
  bcol = jnp.pad(coords[:, 0], (0, pad))
  xcol = jnp.pad(coords[:, 1], (0, pad))
  ycol = jnp.pad(coords[:, 2], (0, pad))
  fext = jnp.concatenate(
      [features, jnp.zeros((648, nin), features.dtype)], axis=0)

  table = _build_table(npad, n, n)(bcol, xcol, ycol)
  tlines, pwords = _tap_lines(npad)(
      table.reshape(TSLOT, 128), bcol, xcol, ycol)
  fix = _extract(npad * 9, 9216, n)(tlines, pwords.reshape(-1, 1))
  g = _gather_rows(npad, nin, n)(fix.reshape(-1), fext)

  bn = 2048
  nt = -(-n // bn) * bn
  out = _tap_matmul(nt, bn, nin, nout)(g, W, bias.reshape(1, nout))
  return out[:n]
